# Initial kernel scaffold; baseline (speedup 1.0000x reference)
import jax
import jax.numpy as jnp
from jax.experimental import pallas as pl


def kernel(hedge_attr, emb_weight):
    raise NotImplementedError("write your pallas kernel here")


# speedup vs baseline: 1.0198x; 1.0198x over previous
import functools

import jax
import jax.numpy as jnp
from jax import lax
from jax.experimental import pallas as pl
from jax.experimental.pallas import tpu as pltpu
from jax.experimental.pallas import tpu_sc as plsc

N_EDGES = 640000
EMB_DIM = 128

_info = plsc.get_sparse_core_info()
NUM_CORES = _info.num_cores
NUM_SUBCORES = _info.num_subcores
NW = NUM_CORES * NUM_SUBCORES
B_PER_W = N_EDGES // NW
CHUNK = 80
N_CHUNKS = B_PER_W // CHUNK


def _sc_body(idx_hbm, table_hbm, out_hbm, idx_v, buf0, buf1,
             gsem0, gsem1, wsem0, wsem1):
    wid = lax.axis_index("s") * NUM_CORES + lax.axis_index("c")
    base = wid * B_PER_W

    pltpu.sync_copy(idx_hbm.at[wid], idx_v)

    def gather(j, buf, sem):
        pltpu.async_copy(table_hbm.at[idx_v.at[j]], buf, sem)

    def write(j, buf, sem):
        pltpu.make_async_copy(
            buf, out_hbm.at[pl.ds(base + j * CHUNK, CHUNK)], sem).start()

    gather(0, buf0, gsem0)
    gather(1, buf1, gsem1)

    def body(k, _):
        c0 = 2 * k
        c1 = 2 * k + 1
        pltpu.make_async_copy(table_hbm.at[idx_v.at[c0]], buf0, gsem0).wait()
        write(c0, buf0, wsem0)
        pltpu.make_async_copy(table_hbm.at[idx_v.at[c1]], buf1, gsem1).wait()
        write(c1, buf1, wsem1)
        pltpu.make_async_copy(
            buf0, out_hbm.at[pl.ds(base + c0 * CHUNK, CHUNK)], wsem0).wait()

        @pl.when(c0 + 2 < N_CHUNKS)
        def _():
            gather(c0 + 2, buf0, gsem0)

        pltpu.make_async_copy(
            buf1, out_hbm.at[pl.ds(base + c1 * CHUNK, CHUNK)], wsem1).wait()

        @pl.when(c1 + 2 < N_CHUNKS)
        def _():
            gather(c1 + 2, buf1, gsem1)

        return 0

    lax.fori_loop(0, N_CHUNKS // 2, body, 0)


@functools.partial(jax.jit, static_argnames=())
def _sc_lookup(idx2d, table):
    mesh = plsc.VectorSubcoreMesh(core_axis_name="c", subcore_axis_name="s")
    f = pl.kernel(
        _sc_body,
        out_type=jax.ShapeDtypeStruct((N_EDGES, EMB_DIM), jnp.float32),
        mesh=mesh,
        scratch_types=[
            pltpu.VMEM((N_CHUNKS, CHUNK), jnp.int32),
            pltpu.VMEM((CHUNK, EMB_DIM), jnp.float32),
            pltpu.VMEM((CHUNK, EMB_DIM), jnp.float32),
            pltpu.SemaphoreType.DMA,
            pltpu.SemaphoreType.DMA,
            pltpu.SemaphoreType.DMA,
            pltpu.SemaphoreType.DMA,
        ],
    )
    return f(idx2d, table)


def kernel(hedge_attr, emb_weight):
    idx3d = hedge_attr.astype(jnp.int32).reshape(NW, N_CHUNKS, CHUNK)
    return _sc_lookup(idx3d, emb_weight)


# speedup vs baseline: 1.0259x; 1.0061x over previous
import functools

import jax
import jax.numpy as jnp
from jax import lax
from jax.experimental import pallas as pl
from jax.experimental.pallas import tpu as pltpu
from jax.experimental.pallas import tpu_sc as plsc

N_EDGES = 640000
EMB_DIM = 128

_info = plsc.get_sparse_core_info()
NUM_CORES = _info.num_cores
NUM_SUBCORES = _info.num_subcores
NW = NUM_CORES * NUM_SUBCORES
B_PER_W = N_EDGES // NW
CHUNK = 80
N_CHUNKS = B_PER_W // CHUNK
NBUF = 5


def _sc_body(idx_hbm, table_hbm, out_hbm, idx_v, bufs, gsems, wsems):
    wid = lax.axis_index("s") * NUM_CORES + lax.axis_index("c")
    base = wid * B_PER_W

    pltpu.sync_copy(idx_hbm.at[wid], idx_v)

    def gather(j, b):
        pltpu.async_copy(table_hbm.at[idx_v.at[j]], bufs[b], gsems[b])

    def wait_gather(j, b):
        pltpu.make_async_copy(table_hbm.at[idx_v.at[j]], bufs[b],
                              gsems[b]).wait()

    def write(j, b):
        pltpu.make_async_copy(
            bufs[b], out_hbm.at[pl.ds(base + j * CHUNK, CHUNK)],
            wsems[b]).start()

    def wait_write(j, b):
        pltpu.make_async_copy(
            bufs[b], out_hbm.at[pl.ds(base + j * CHUNK, CHUNK)],
            wsems[b]).wait()

    for b in range(NBUF):
        gather(b, b)

    def body(k, _):
        c = k * NBUF
        for b in range(NBUF):
            wait_gather(c + b, b)
            write(c + b, b)
        for b in range(NBUF):
            wait_write(c + b, b)

            @pl.when(c + b + NBUF < N_CHUNKS)
            def _():
                gather(c + b + NBUF, b)

        return 0

    lax.fori_loop(0, N_CHUNKS // NBUF, body, 0)


@functools.partial(jax.jit, static_argnames=())
def _sc_lookup(idx2d, table):
    mesh = plsc.VectorSubcoreMesh(core_axis_name="c", subcore_axis_name="s")
    f = pl.kernel(
        _sc_body,
        out_type=jax.ShapeDtypeStruct((N_EDGES, EMB_DIM), jnp.float32),
        mesh=mesh,
        scratch_types=[
            pltpu.VMEM((N_CHUNKS, CHUNK), jnp.int32),
            [pltpu.VMEM((CHUNK, EMB_DIM), jnp.float32) for _ in range(NBUF)],
            [pltpu.SemaphoreType.DMA for _ in range(NBUF)],
            [pltpu.SemaphoreType.DMA for _ in range(NBUF)],
        ],
    )
    return f(idx2d, table)


def kernel(hedge_attr, emb_weight):
    idx3d = hedge_attr.astype(jnp.int32).reshape(NW, N_CHUNKS, CHUNK)
    return _sc_lookup(idx3d, emb_weight)


# speedup vs baseline: 15.8378x; 15.4372x over previous
import functools

import jax
import jax.numpy as jnp
from jax import lax
from jax.experimental import pallas as pl
from jax.experimental.pallas import tpu as pltpu
from jax.experimental.pallas import tpu_sc as plsc

N_EDGES = 640000
EMB_DIM = 128

_info = plsc.get_sparse_core_info()
NUM_CORES = _info.num_cores
NUM_SUBCORES = _info.num_subcores
NW = NUM_CORES * NUM_SUBCORES
B_PER_W = N_EDGES // NW
CHUNK = 80
N_CHUNKS = B_PER_W // CHUNK
NBUF = 5


def _sc_body(idx_hbm, table_hbm, out_hbm, table_sh, idx_v, bufs,
             gsems, wsems):
    sid = lax.axis_index("s")
    wid = sid * NUM_CORES + lax.axis_index("c")
    base = wid * B_PER_W

    @pl.when(sid == 0)
    def _():
        pltpu.sync_copy(table_hbm, table_sh)

    pltpu.sync_copy(idx_hbm.at[wid], idx_v)
    plsc.subcore_barrier()

    def gather(j, b):
        pltpu.async_copy(table_sh.at[idx_v.at[j]], bufs[b], gsems[b])

    def wait_gather(j, b):
        pltpu.make_async_copy(table_sh.at[idx_v.at[j]], bufs[b],
                              gsems[b]).wait()

    def write(j, b):
        pltpu.make_async_copy(
            bufs[b], out_hbm.at[pl.ds(base + j * CHUNK, CHUNK)],
            wsems[b]).start()

    def wait_write(j, b):
        pltpu.make_async_copy(
            bufs[b], out_hbm.at[pl.ds(base + j * CHUNK, CHUNK)],
            wsems[b]).wait()

    for b in range(NBUF):
        gather(b, b)

    def body(k, _):
        c = k * NBUF
        for b in range(NBUF):
            wait_gather(c + b, b)
            write(c + b, b)
        for b in range(NBUF):
            wait_write(c + b, b)

            @pl.when(c + b + NBUF < N_CHUNKS)
            def _():
                gather(c + b + NBUF, b)

        return 0

    lax.fori_loop(0, N_CHUNKS // NBUF, body, 0)


@functools.partial(jax.jit, static_argnames=())
def _sc_lookup(idx2d, table):
    mesh = plsc.VectorSubcoreMesh(core_axis_name="c", subcore_axis_name="s")
    f = pl.kernel(
        _sc_body,
        out_type=jax.ShapeDtypeStruct((N_EDGES, EMB_DIM), jnp.float32),
        mesh=mesh,
        scratch_types=[
            pltpu.VMEM_SHARED((14, EMB_DIM), jnp.float32),
            pltpu.VMEM((N_CHUNKS, CHUNK), jnp.int32),
            [pltpu.VMEM((CHUNK, EMB_DIM), jnp.float32) for _ in range(NBUF)],
            [pltpu.SemaphoreType.DMA for _ in range(NBUF)],
            [pltpu.SemaphoreType.DMA for _ in range(NBUF)],
        ],
    )
    return f(idx2d, table)


def kernel(hedge_attr, emb_weight):
    idx3d = hedge_attr.astype(jnp.int32).reshape(NW, N_CHUNKS, CHUNK)
    return _sc_lookup(idx3d, emb_weight)


# speedup vs baseline: 15.8934x; 1.0035x over previous
import functools

import jax
import jax.numpy as jnp
from jax import lax
from jax.experimental import pallas as pl
from jax.experimental.pallas import tpu as pltpu
from jax.experimental.pallas import tpu_sc as plsc

N_EDGES = 640000
EMB_DIM = 128

_info = plsc.get_sparse_core_info()
NUM_CORES = _info.num_cores
NUM_SUBCORES = _info.num_subcores
NW = NUM_CORES * NUM_SUBCORES
B_PER_W = N_EDGES // NW
CHUNK = 80
N_CHUNKS = B_PER_W // CHUNK
GPS = 5
SUP = CHUNK * GPS
N_SUP = B_PER_W // SUP


def _sc_body(idx_hbm, table_hbm, out_hbm, table_sh, idx_v,
             buf0, buf1, g0, g1, w0, w1):
    sid = lax.axis_index("s")
    wid = sid * NUM_CORES + lax.axis_index("c")
    base = wid * B_PER_W

    @pl.when(sid == 0)
    def _():
        pltpu.sync_copy(table_hbm, table_sh)

    pltpu.sync_copy(idx_hbm.at[pl.ds(base, B_PER_W)], idx_v)
    plsc.subcore_barrier()

    def fire_gathers(s, buf, gsem):
        for i in range(GPS):
            pltpu.async_copy(
                table_sh.at[idx_v.at[pl.ds((s * GPS + i) * CHUNK, CHUNK)]],
                buf.at[pl.ds(i * CHUNK, CHUNK)], gsem)

    def wait_gathers(s, buf, gsem):
        for i in range(GPS):
            pltpu.make_async_copy(
                table_sh.at[idx_v.at[pl.ds((s * GPS + i) * CHUNK, CHUNK)]],
                buf.at[pl.ds(i * CHUNK, CHUNK)], gsem).wait()

    def write(s, buf, wsem):
        pltpu.make_async_copy(
            buf, out_hbm.at[pl.ds(base + s * SUP, SUP)], wsem).start()

    def wait_write(s, buf, wsem):
        pltpu.make_async_copy(
            buf, out_hbm.at[pl.ds(base + s * SUP, SUP)], wsem).wait()

    fire_gathers(0, buf0, g0)

    def body(k, _):
        s0 = 2 * k
        s1 = 2 * k + 1
        wait_gathers(s0, buf0, g0)
        write(s0, buf0, w0)

        @pl.when(k > 0)
        def _():
            wait_write(s1 - 2, buf1, w1)

        fire_gathers(s1, buf1, g1)
        wait_gathers(s1, buf1, g1)
        write(s1, buf1, w1)
        wait_write(s0, buf0, w0)

        @pl.when(s0 + 2 < N_SUP)
        def _():
            fire_gathers(s0 + 2, buf0, g0)

        return 0

    lax.fori_loop(0, N_SUP // 2, body, 0)
    wait_write(N_SUP - 1, buf1, w1)


@functools.partial(jax.jit, static_argnames=())
def _sc_lookup(idx2d, table):
    mesh = plsc.VectorSubcoreMesh(core_axis_name="c", subcore_axis_name="s")
    f = pl.kernel(
        _sc_body,
        out_type=jax.ShapeDtypeStruct((N_EDGES, EMB_DIM), jnp.float32),
        mesh=mesh,
        scratch_types=[
            pltpu.VMEM_SHARED((14, EMB_DIM), jnp.float32),
            pltpu.VMEM((B_PER_W,), jnp.int32),
            pltpu.VMEM((SUP, EMB_DIM), jnp.float32),
            pltpu.VMEM((SUP, EMB_DIM), jnp.float32),
            pltpu.SemaphoreType.DMA,
            pltpu.SemaphoreType.DMA,
            pltpu.SemaphoreType.DMA,
            pltpu.SemaphoreType.DMA,
        ],
    )
    return f(idx2d, table)


def kernel(hedge_attr, emb_weight):
    return _sc_lookup(hedge_attr.astype(jnp.int32), emb_weight)


# speedup vs baseline: 15.9195x; 1.0016x over previous
import functools

import jax
import jax.numpy as jnp
from jax import lax
from jax.experimental import pallas as pl
from jax.experimental.pallas import tpu as pltpu
from jax.experimental.pallas import tpu_sc as plsc

N_EDGES = 640000
EMB_DIM = 128

_info = plsc.get_sparse_core_info()
NUM_CORES = _info.num_cores
NUM_SUBCORES = _info.num_subcores
NW = NUM_CORES * NUM_SUBCORES
B_PER_W = N_EDGES // NW
CHUNK = 80
N_CHUNKS = B_PER_W // CHUNK
GPS = 5
SUP = CHUNK * GPS
N_SUP = B_PER_W // SUP


def _sc_body(idx_hbm, table_hbm, out_hbm, table_sh, idx_v,
             buf0, buf1, g0, g1, w0, w1):
    sid = lax.axis_index("s")
    wid = sid * NUM_CORES + lax.axis_index("c")
    base = wid * B_PER_W

    @pl.when(sid == 0)
    def _():
        pltpu.sync_copy(table_hbm, table_sh)

    pltpu.sync_copy(idx_hbm.at[pl.ds(base, B_PER_W)], idx_v)
    plsc.subcore_barrier()

    def fire_gathers(s, buf, gsem):
        for i in range(GPS):
            pltpu.async_copy(
                table_sh.at[idx_v.at[pl.ds((s * GPS + i) * CHUNK, CHUNK)]],
                buf.at[pl.ds(i * CHUNK, CHUNK)], gsem)

    def wait_gathers(s, buf, gsem):
        for i in range(GPS):
            pltpu.make_async_copy(
                table_sh.at[idx_v.at[pl.ds((s * GPS + i) * CHUNK, CHUNK)]],
                buf.at[pl.ds(i * CHUNK, CHUNK)], gsem).wait()

    def write(s, buf, wsem):
        pltpu.make_async_copy(
            buf, out_hbm.at[pl.ds(base + s * SUP, SUP)], wsem).start()

    def wait_write(s, buf, wsem):
        pltpu.make_async_copy(
            buf, out_hbm.at[pl.ds(base + s * SUP, SUP)], wsem).wait()

    fire_gathers(0, buf0, g0)

    def body(k, _):
        s0 = 2 * k
        s1 = 2 * k + 1
        wait_gathers(s0, buf0, g0)
        write(s0, buf0, w0)

        @pl.when(k > 0)
        def _():
            wait_write(s1 - 2, buf1, w1)

        fire_gathers(s1, buf1, g1)
        wait_gathers(s1, buf1, g1)
        write(s1, buf1, w1)
        wait_write(s0, buf0, w0)

        @pl.when(s0 + 2 < N_SUP)
        def _():
            fire_gathers(s0 + 2, buf0, g0)

        return 0

    lax.fori_loop(0, N_SUP // 2, body, 0)
    wait_write(N_SUP - 1, buf1, w1)


@functools.partial(jax.jit, static_argnames=())
def _sc_lookup(idx, table):
    mesh = plsc.VectorSubcoreMesh(core_axis_name="c", subcore_axis_name="s")
    f = pl.kernel(
        _sc_body,
        out_type=jax.ShapeDtypeStruct((N_EDGES, EMB_DIM), jnp.float32),
        mesh=mesh,
        scratch_types=[
            pltpu.VMEM_SHARED((14, EMB_DIM), jnp.float32),
            pltpu.VMEM((B_PER_W,), jnp.int32),
            pltpu.VMEM((SUP, EMB_DIM), jnp.float32),
            pltpu.VMEM((SUP, EMB_DIM), jnp.float32),
            pltpu.SemaphoreType.DMA,
            pltpu.SemaphoreType.DMA,
            pltpu.SemaphoreType.DMA,
            pltpu.SemaphoreType.DMA,
        ],
    )
    return f(idx, table)


def kernel(hedge_attr, emb_weight):
    return _sc_lookup(hedge_attr.astype(jnp.int32), emb_weight)
